# baseline (speedup 1.0000x reference)
import functools

import jax
import jax.numpy as jnp
from jax import lax
from jax.experimental import pallas as pl
from jax.experimental.pallas import tpu as pltpu
from jax.experimental.pallas import tpu_sc as plsc

B = 16384
DG = 32
DM = 32
H1, H2, H3 = 64, 32, 16
NROW = 1000000

NC, NS = 2, 16
NW = NC * NS
BPW = B // NW
NSLOT = 2
GRP = 16
NGRP = BPW // GRP


def _sc_gather(uhi, ulo, ihi, ilo, gut, git, mut, mit):
    mesh = plsc.VectorSubcoreMesh(core_axis_name="c", subcore_axis_name="s")
    row = jax.ShapeDtypeStruct((B, DG), jnp.float32)
    blk = pltpu.VMEM((NSLOT * DG, 128), jnp.float32)
    rbuf = pltpu.VMEM((GRP, DG), jnp.float32)

    @functools.partial(
        pl.kernel,
        out_type=(row, row, row),
        mesh=mesh,
        compiler_params=pltpu.CompilerParams(needs_layout_passes=False),
        scratch_types=[
            pltpu.VMEM((BPW,), jnp.int32),
            pltpu.VMEM((BPW,), jnp.int32),
            pltpu.VMEM((BPW,), jnp.int32),
            pltpu.VMEM((BPW,), jnp.int32),
            blk, blk, blk, blk,
            rbuf, rbuf, rbuf,
            pltpu.SemaphoreType.DMA,
            pltpu.SemaphoreType.DMA,
        ],
    )
    def gather_kernel(uhi_h, ulo_h, ihi_h, ilo_h, gu_h, gi_h, mu_h, mi_h,
                      out_gmf, out_mu, out_mi,
                      thi, tlo, shi, slo, bgu, bgi, bmu, bmi,
                      rg, rmu, rmi, semA, semB):
        wid = lax.axis_index("s") * NC + lax.axis_index("c")
        base = wid * BPW
        pltpu.sync_copy(uhi_h.at[pl.ds(base, BPW)], thi)
        pltpu.sync_copy(ulo_h.at[pl.ds(base, BPW)], tlo)
        pltpu.sync_copy(ihi_h.at[pl.ds(base, BPW)], shi)
        pltpu.sync_copy(ilo_h.at[pl.ds(base, BPW)], slo)
        iota = lax.broadcasted_iota(jnp.int32, (16,), 0)

        def group(g, carry):
            gbase = g * GRP
            uv = thi[pl.ds(gbase, GRP)]
            iv = shi[pl.ds(gbase, GRP)]
            ulv = tlo[pl.ds(gbase, GRP)]
            ilv = slo[pl.ds(gbase, GRP)]
            sems = [semA, semB]

            def fire(n, slot):
                tu = uv[n] * 128
                ti = iv[n] * 128
                dst = pl.ds(slot * DG, DG)
                sem = sems[slot]
                return [
                    pltpu.async_copy(gu_h.at[:, pl.ds(tu, 128)], bgu.at[dst], sem),
                    pltpu.async_copy(gi_h.at[:, pl.ds(ti, 128)], bgi.at[dst], sem),
                    pltpu.async_copy(mu_h.at[:, pl.ds(tu, 128)], bmu.at[dst], sem),
                    pltpu.async_copy(mi_h.at[:, pl.ds(ti, 128)], bmi.at[dst], sem),
                ]

            inflight = {0: fire(0, 0), 1: fire(1, 1)}
            for n in range(GRP):
                slot = n % NSLOT
                for c in inflight[slot]:
                    c.wait()
                lu = iota * 0 + ulv[n]
                li = iota * 0 + ilv[n]
                ra = iota + slot * DG
                rb = ra + 16
                g0 = plsc.load_gather(bgu, [ra, lu])
                g1 = plsc.load_gather(bgu, [rb, lu])
                h0 = plsc.load_gather(bgi, [ra, li])
                h1 = plsc.load_gather(bgi, [rb, li])
                m0 = plsc.load_gather(bmu, [ra, lu])
                m1 = plsc.load_gather(bmu, [rb, lu])
                n0 = plsc.load_gather(bmi, [ra, li])
                n1 = plsc.load_gather(bmi, [rb, li])
                lo, hi = pl.ds(0, 16), pl.ds(16, 16)
                rg[n, lo] = g0 * h0
                rg[n, hi] = g1 * h1
                rmu[n, lo] = m0
                rmu[n, hi] = m1
                rmi[n, lo] = n0
                rmi[n, hi] = n1
                if n + NSLOT < GRP:
                    inflight[slot] = fire(n + NSLOT, slot)

            osl = pl.ds(base + gbase, GRP)
            pltpu.sync_copy(rg, out_gmf.at[osl])
            pltpu.sync_copy(rmu, out_mu.at[osl])
            pltpu.sync_copy(rmi, out_mi.at[osl])
            return carry

        lax.fori_loop(0, NGRP, group, 0)

    return gather_kernel(uhi, ulo, ihi, ilo, gut, git, mut, mit)


BLK = 1024
GRID = B // BLK


def _mlp_body(gmf_r, emu, emi, lab,
              w1a, w1b, b1, w2, b2, w3, b3, wpa, wpb, bp,
              m1_r, m2_r, m3_r, pred_r, loss_r):
    i = pl.program_id(0)
    dot = functools.partial(
        lax.dot_general,
        dimension_numbers=(((1,), (1,)), ((), ())),
        preferred_element_type=jnp.float32,
    )
    m1 = jnp.maximum(dot(emu[...], w1a[...]) + dot(emi[...], w1b[...]) + b1[...], 0.0)
    m2 = jnp.maximum(dot(m1, w2[...]) + b2[...], 0.0)
    m3 = jnp.maximum(dot(m2, w3[...]) + b3[...], 0.0)
    gmf = gmf_r[...]
    s = dot(gmf, wpa[...]) + dot(m3, wpb[...]) + bp[...]
    pred = jax.nn.sigmoid(s)
    m1_r[...] = m1
    m2_r[...] = m2
    m3_r[...] = m3
    pred_r[...] = pred
    p = jnp.clip(pred, 1e-7, 1.0 - 1e-7)
    y = lab[...].astype(jnp.float32)
    part = jnp.sum(-(y * jnp.log(p) + (1.0 - y) * jnp.log(1.0 - p)))

    @pl.when(i == 0)
    def _():
        loss_r[0, 0] = part

    @pl.when(i > 0)
    def _():
        loss_r[0, 0] += part

    @pl.when(i == GRID - 1)
    def _():
        loss_r[0, 0] = loss_r[0, 0] / B


def _tc_mlp(gmf, emu, emi, lab2, W1a, W1b, b1, W2, b2, W3, b3, Wpa, Wpb, bp):
    bspec = lambda d: pl.BlockSpec((BLK, d), lambda i: (i, 0))
    wspec = lambda r, c: pl.BlockSpec((r, c), lambda i: (0, 0))
    return pl.pallas_call(
        _mlp_body,
        grid=(GRID,),
        in_specs=[
            bspec(DG), bspec(DM), bspec(DM), bspec(1),
            wspec(H1, DM), wspec(H1, DM), wspec(1, H1),
            wspec(H2, H1), wspec(1, H2),
            wspec(H3, H2), wspec(1, H3),
            wspec(1, DG), wspec(1, H3), wspec(1, 1),
        ],
        out_specs=[
            bspec(H1), bspec(H2), bspec(H3), bspec(1),
            pl.BlockSpec(memory_space=pltpu.SMEM, block_shape=(1, 1),
                         index_map=lambda i: (0, 0)),
        ],
        out_shape=[
            jax.ShapeDtypeStruct((B, H1), jnp.float32),
            jax.ShapeDtypeStruct((B, H2), jnp.float32),
            jax.ShapeDtypeStruct((B, H3), jnp.float32),
            jax.ShapeDtypeStruct((B, 1), jnp.float32),
            jax.ShapeDtypeStruct((1, 1), jnp.float32),
        ],
    )(gmf, emu, emi, lab2, W1a, W1b, b1, W2, b2, W3, b3, Wpa, Wpb, bp)


def kernel(user, item, label, eu_gmf, ei_gmf, eu_mlp, ei_mlp,
           W1, b1, W2, b2, W3, b3, Wp, bp):
    uhi = lax.shift_right_logical(user, 7)
    ulo = jnp.bitwise_and(user, 127)
    ihi = lax.shift_right_logical(item, 7)
    ilo = jnp.bitwise_and(item, 127)
    gmf, emu, emi = _sc_gather(uhi, ulo, ihi, ilo,
                               eu_gmf.T, ei_gmf.T, eu_mlp.T, ei_mlp.T)
    W1a, W1b = W1[:, :DM], W1[:, DM:]
    Wpa, Wpb = Wp[:, :DG], Wp[:, DG:]
    m1, m2, m3, pred2, loss = _tc_mlp(
        gmf, emu, emi, label.reshape(B, 1),
        W1a, W1b, b1.reshape(1, H1), W2, b2.reshape(1, H2),
        W3, b3.reshape(1, H3), Wpa, Wpb, bp.reshape(1, 1),
    )
    return (loss[0, 0], m1, m2, m3, pred2.reshape(-1))


# speedup vs baseline: 1.1236x; 1.1236x over previous
import functools

import jax
import jax.numpy as jnp
from jax import lax
from jax.experimental import pallas as pl
from jax.experimental.pallas import tpu as pltpu
from jax.experimental.pallas import tpu_sc as plsc

B = 16384
DG = 32
DM = 32
H1, H2, H3 = 64, 32, 16
NROW = 1000000

NC, NS = 2, 16
NW = NC * NS
BPW = B // NW
NSLOT = 4
GRP = 16
NGRP = BPW // GRP


def _sc_gather(uhi, ulo, ihi, ilo, gut, git, mut, mit):
    mesh = plsc.VectorSubcoreMesh(core_axis_name="c", subcore_axis_name="s")
    row = jax.ShapeDtypeStruct((B, DG), jnp.float32)
    blk = pltpu.VMEM((NSLOT * DG, 128), jnp.float32)
    rbuf = pltpu.VMEM((GRP, DG), jnp.float32)

    @functools.partial(
        pl.kernel,
        out_type=(row, row, row),
        mesh=mesh,
        compiler_params=pltpu.CompilerParams(needs_layout_passes=False),
        scratch_types=[
            pltpu.VMEM((BPW,), jnp.int32),
            pltpu.VMEM((BPW,), jnp.int32),
            pltpu.VMEM((BPW,), jnp.int32),
            pltpu.VMEM((BPW,), jnp.int32),
            blk, blk, blk, blk,
            rbuf, rbuf, rbuf,
            pltpu.SemaphoreType.DMA,
            pltpu.SemaphoreType.DMA,
            pltpu.SemaphoreType.DMA,
            pltpu.SemaphoreType.DMA,
        ],
    )
    def gather_kernel(uhi_h, ulo_h, ihi_h, ilo_h, gu_h, gi_h, mu_h, mi_h,
                      out_gmf, out_mu, out_mi,
                      thi, tlo, shi, slo, bgu, bgi, bmu, bmi,
                      rg, rmu, rmi, semA, semB, semC, semD):
        wid = lax.axis_index("s") * NC + lax.axis_index("c")
        base = wid * BPW
        pltpu.sync_copy(uhi_h.at[pl.ds(base, BPW)], thi)
        pltpu.sync_copy(ulo_h.at[pl.ds(base, BPW)], tlo)
        pltpu.sync_copy(ihi_h.at[pl.ds(base, BPW)], shi)
        pltpu.sync_copy(ilo_h.at[pl.ds(base, BPW)], slo)
        iota = lax.broadcasted_iota(jnp.int32, (16,), 0)

        def group(g, carry):
            gbase = g * GRP
            uv = thi[pl.ds(gbase, GRP)]
            iv = shi[pl.ds(gbase, GRP)]
            ulv = tlo[pl.ds(gbase, GRP)]
            ilv = slo[pl.ds(gbase, GRP)]
            sems = [semA, semB, semC, semD]

            def fire(n, slot):
                tu = uv[n] * 128
                ti = iv[n] * 128
                dst = pl.ds(slot * DG, DG)
                sem = sems[slot]
                return [
                    pltpu.async_copy(gu_h.at[:, pl.ds(tu, 128)], bgu.at[dst], sem),
                    pltpu.async_copy(gi_h.at[:, pl.ds(ti, 128)], bgi.at[dst], sem),
                    pltpu.async_copy(mu_h.at[:, pl.ds(tu, 128)], bmu.at[dst], sem),
                    pltpu.async_copy(mi_h.at[:, pl.ds(ti, 128)], bmi.at[dst], sem),
                ]

            inflight = {s: fire(s, s) for s in range(NSLOT)}
            for n in range(GRP):
                slot = n % NSLOT
                for c in inflight[slot]:
                    c.wait()
                lu = iota * 0 + ulv[n]
                li = iota * 0 + ilv[n]
                ra = iota + slot * DG
                rb = ra + 16
                g0 = plsc.load_gather(bgu, [ra, lu])
                g1 = plsc.load_gather(bgu, [rb, lu])
                h0 = plsc.load_gather(bgi, [ra, li])
                h1 = plsc.load_gather(bgi, [rb, li])
                m0 = plsc.load_gather(bmu, [ra, lu])
                m1 = plsc.load_gather(bmu, [rb, lu])
                n0 = plsc.load_gather(bmi, [ra, li])
                n1 = plsc.load_gather(bmi, [rb, li])
                lo, hi = pl.ds(0, 16), pl.ds(16, 16)
                rg[n, lo] = g0 * h0
                rg[n, hi] = g1 * h1
                rmu[n, lo] = m0
                rmu[n, hi] = m1
                rmi[n, lo] = n0
                rmi[n, hi] = n1
                if n + NSLOT < GRP:
                    inflight[slot] = fire(n + NSLOT, slot)

            osl = pl.ds(base + gbase, GRP)
            pltpu.sync_copy(rg, out_gmf.at[osl])
            pltpu.sync_copy(rmu, out_mu.at[osl])
            pltpu.sync_copy(rmi, out_mi.at[osl])
            return carry

        lax.fori_loop(0, NGRP, group, 0)

    return gather_kernel(uhi, ulo, ihi, ilo, gut, git, mut, mit)


BLK = 1024
GRID = B // BLK


def _mlp_body(gmf_r, emu, emi, lab,
              w1a, w1b, b1, w2, b2, w3, b3, wpa, wpb, bp,
              m1_r, m2_r, m3_r, pred_r, loss_r):
    i = pl.program_id(0)
    dot = functools.partial(
        lax.dot_general,
        dimension_numbers=(((1,), (1,)), ((), ())),
        preferred_element_type=jnp.float32,
    )
    m1 = jnp.maximum(dot(emu[...], w1a[...]) + dot(emi[...], w1b[...]) + b1[...], 0.0)
    m2 = jnp.maximum(dot(m1, w2[...]) + b2[...], 0.0)
    m3 = jnp.maximum(dot(m2, w3[...]) + b3[...], 0.0)
    gmf = gmf_r[...]
    s = dot(gmf, wpa[...]) + dot(m3, wpb[...]) + bp[...]
    pred = jax.nn.sigmoid(s)
    m1_r[...] = m1
    m2_r[...] = m2
    m3_r[...] = m3
    pred_r[...] = pred
    p = jnp.clip(pred, 1e-7, 1.0 - 1e-7)
    y = lab[...].astype(jnp.float32)
    part = jnp.sum(-(y * jnp.log(p) + (1.0 - y) * jnp.log(1.0 - p)))

    @pl.when(i == 0)
    def _():
        loss_r[0, 0] = part

    @pl.when(i > 0)
    def _():
        loss_r[0, 0] += part

    @pl.when(i == GRID - 1)
    def _():
        loss_r[0, 0] = loss_r[0, 0] / B


def _tc_mlp(gmf, emu, emi, lab2, W1a, W1b, b1, W2, b2, W3, b3, Wpa, Wpb, bp):
    bspec = lambda d: pl.BlockSpec((BLK, d), lambda i: (i, 0))
    wspec = lambda r, c: pl.BlockSpec((r, c), lambda i: (0, 0))
    return pl.pallas_call(
        _mlp_body,
        grid=(GRID,),
        in_specs=[
            bspec(DG), bspec(DM), bspec(DM), bspec(1),
            wspec(H1, DM), wspec(H1, DM), wspec(1, H1),
            wspec(H2, H1), wspec(1, H2),
            wspec(H3, H2), wspec(1, H3),
            wspec(1, DG), wspec(1, H3), wspec(1, 1),
        ],
        out_specs=[
            bspec(H1), bspec(H2), bspec(H3), bspec(1),
            pl.BlockSpec(memory_space=pltpu.SMEM, block_shape=(1, 1),
                         index_map=lambda i: (0, 0)),
        ],
        out_shape=[
            jax.ShapeDtypeStruct((B, H1), jnp.float32),
            jax.ShapeDtypeStruct((B, H2), jnp.float32),
            jax.ShapeDtypeStruct((B, H3), jnp.float32),
            jax.ShapeDtypeStruct((B, 1), jnp.float32),
            jax.ShapeDtypeStruct((1, 1), jnp.float32),
        ],
    )(gmf, emu, emi, lab2, W1a, W1b, b1, W2, b2, W3, b3, Wpa, Wpb, bp)


def kernel(user, item, label, eu_gmf, ei_gmf, eu_mlp, ei_mlp,
           W1, b1, W2, b2, W3, b3, Wp, bp):
    uhi = lax.shift_right_logical(user, 7)
    ulo = jnp.bitwise_and(user, 127)
    ihi = lax.shift_right_logical(item, 7)
    ilo = jnp.bitwise_and(item, 127)
    gmf, emu, emi = _sc_gather(uhi, ulo, ihi, ilo,
                               eu_gmf.T, ei_gmf.T, eu_mlp.T, ei_mlp.T)
    W1a, W1b = W1[:, :DM], W1[:, DM:]
    Wpa, Wpb = Wp[:, :DG], Wp[:, DG:]
    m1, m2, m3, pred2, loss = _tc_mlp(
        gmf, emu, emi, label.reshape(B, 1),
        W1a, W1b, b1.reshape(1, H1), W2, b2.reshape(1, H2),
        W3, b3.reshape(1, H3), Wpa, Wpb, bp.reshape(1, 1),
    )
    return (loss[0, 0], m1, m2, m3, pred2.reshape(-1))


# speedup vs baseline: 1.1557x; 1.0285x over previous
import functools

import jax
import jax.numpy as jnp
from jax import lax
from jax.experimental import pallas as pl
from jax.experimental.pallas import tpu as pltpu
from jax.experimental.pallas import tpu_sc as plsc

B = 16384
DG = 32
DM = 32
H1, H2, H3 = 64, 32, 16
NROW = 1000000

NC, NS = 2, 16
NW = NC * NS
BPW = B // NW
NSLOT = 6
GRP = 16
NGRP = BPW // GRP


def _sc_gather(uhi, ulo, ihi, ilo, gut, git, mut, mit):
    mesh = plsc.VectorSubcoreMesh(core_axis_name="c", subcore_axis_name="s")
    row = jax.ShapeDtypeStruct((B, DG), jnp.float32)
    blk = pltpu.VMEM((NSLOT * DG, 128), jnp.float32)
    rbuf = pltpu.VMEM((GRP, DG), jnp.float32)

    @functools.partial(
        pl.kernel,
        out_type=(row, row, row),
        mesh=mesh,
        compiler_params=pltpu.CompilerParams(needs_layout_passes=False),
        scratch_types=[
            pltpu.VMEM((BPW,), jnp.int32),
            pltpu.VMEM((BPW,), jnp.int32),
            pltpu.VMEM((BPW,), jnp.int32),
            pltpu.VMEM((BPW,), jnp.int32),
            blk, blk, blk, blk,
            rbuf, rbuf, rbuf,
            pltpu.SemaphoreType.DMA,
            pltpu.SemaphoreType.DMA,
            pltpu.SemaphoreType.DMA,
            pltpu.SemaphoreType.DMA,
            pltpu.SemaphoreType.DMA,
            pltpu.SemaphoreType.DMA,
        ],
    )
    def gather_kernel(uhi_h, ulo_h, ihi_h, ilo_h, gu_h, gi_h, mu_h, mi_h,
                      out_gmf, out_mu, out_mi,
                      thi, tlo, shi, slo, bgu, bgi, bmu, bmi,
                      rg, rmu, rmi, semA, semB, semC, semD, semE, semF):
        wid = lax.axis_index("s") * NC + lax.axis_index("c")
        base = wid * BPW
        pltpu.sync_copy(uhi_h.at[pl.ds(base, BPW)], thi)
        pltpu.sync_copy(ulo_h.at[pl.ds(base, BPW)], tlo)
        pltpu.sync_copy(ihi_h.at[pl.ds(base, BPW)], shi)
        pltpu.sync_copy(ilo_h.at[pl.ds(base, BPW)], slo)
        iota = lax.broadcasted_iota(jnp.int32, (16,), 0)

        def group(g, carry):
            gbase = g * GRP
            uv = thi[pl.ds(gbase, GRP)]
            iv = shi[pl.ds(gbase, GRP)]
            ulv = tlo[pl.ds(gbase, GRP)]
            ilv = slo[pl.ds(gbase, GRP)]
            sems = [semA, semB, semC, semD, semE, semF]

            def fire(n, slot):
                tu = uv[n] * 128
                ti = iv[n] * 128
                dst = pl.ds(slot * DG, DG)
                sem = sems[slot]
                return [
                    pltpu.async_copy(gu_h.at[:, pl.ds(tu, 128)], bgu.at[dst], sem),
                    pltpu.async_copy(gi_h.at[:, pl.ds(ti, 128)], bgi.at[dst], sem),
                    pltpu.async_copy(mu_h.at[:, pl.ds(tu, 128)], bmu.at[dst], sem),
                    pltpu.async_copy(mi_h.at[:, pl.ds(ti, 128)], bmi.at[dst], sem),
                ]

            inflight = {s: fire(s, s) for s in range(NSLOT)}
            for n in range(GRP):
                slot = n % NSLOT
                for c in inflight[slot]:
                    c.wait()
                lu = iota * 0 + ulv[n]
                li = iota * 0 + ilv[n]
                ra = iota + slot * DG
                rb = ra + 16
                g0 = plsc.load_gather(bgu, [ra, lu])
                g1 = plsc.load_gather(bgu, [rb, lu])
                h0 = plsc.load_gather(bgi, [ra, li])
                h1 = plsc.load_gather(bgi, [rb, li])
                m0 = plsc.load_gather(bmu, [ra, lu])
                m1 = plsc.load_gather(bmu, [rb, lu])
                n0 = plsc.load_gather(bmi, [ra, li])
                n1 = plsc.load_gather(bmi, [rb, li])
                lo, hi = pl.ds(0, 16), pl.ds(16, 16)
                rg[n, lo] = g0 * h0
                rg[n, hi] = g1 * h1
                rmu[n, lo] = m0
                rmu[n, hi] = m1
                rmi[n, lo] = n0
                rmi[n, hi] = n1
                if n + NSLOT < GRP:
                    inflight[slot] = fire(n + NSLOT, slot)

            osl = pl.ds(base + gbase, GRP)
            pltpu.sync_copy(rg, out_gmf.at[osl])
            pltpu.sync_copy(rmu, out_mu.at[osl])
            pltpu.sync_copy(rmi, out_mi.at[osl])
            return carry

        lax.fori_loop(0, NGRP, group, 0)

    return gather_kernel(uhi, ulo, ihi, ilo, gut, git, mut, mit)


BLK = 1024
GRID = B // BLK


def _mlp_body(gmf_r, emu, emi, lab,
              w1a, w1b, b1, w2, b2, w3, b3, wpa, wpb, bp,
              m1_r, m2_r, m3_r, pred_r, loss_r):
    i = pl.program_id(0)
    dot = functools.partial(
        lax.dot_general,
        dimension_numbers=(((1,), (1,)), ((), ())),
        preferred_element_type=jnp.float32,
    )
    m1 = jnp.maximum(dot(emu[...], w1a[...]) + dot(emi[...], w1b[...]) + b1[...], 0.0)
    m2 = jnp.maximum(dot(m1, w2[...]) + b2[...], 0.0)
    m3 = jnp.maximum(dot(m2, w3[...]) + b3[...], 0.0)
    gmf = gmf_r[...]
    s = dot(gmf, wpa[...]) + dot(m3, wpb[...]) + bp[...]
    pred = jax.nn.sigmoid(s)
    m1_r[...] = m1
    m2_r[...] = m2
    m3_r[...] = m3
    pred_r[...] = pred
    p = jnp.clip(pred, 1e-7, 1.0 - 1e-7)
    y = lab[...].astype(jnp.float32)
    part = jnp.sum(-(y * jnp.log(p) + (1.0 - y) * jnp.log(1.0 - p)))

    @pl.when(i == 0)
    def _():
        loss_r[0, 0] = part

    @pl.when(i > 0)
    def _():
        loss_r[0, 0] += part

    @pl.when(i == GRID - 1)
    def _():
        loss_r[0, 0] = loss_r[0, 0] / B


def _tc_mlp(gmf, emu, emi, lab2, W1a, W1b, b1, W2, b2, W3, b3, Wpa, Wpb, bp):
    bspec = lambda d: pl.BlockSpec((BLK, d), lambda i: (i, 0))
    wspec = lambda r, c: pl.BlockSpec((r, c), lambda i: (0, 0))
    return pl.pallas_call(
        _mlp_body,
        grid=(GRID,),
        in_specs=[
            bspec(DG), bspec(DM), bspec(DM), bspec(1),
            wspec(H1, DM), wspec(H1, DM), wspec(1, H1),
            wspec(H2, H1), wspec(1, H2),
            wspec(H3, H2), wspec(1, H3),
            wspec(1, DG), wspec(1, H3), wspec(1, 1),
        ],
        out_specs=[
            bspec(H1), bspec(H2), bspec(H3), bspec(1),
            pl.BlockSpec(memory_space=pltpu.SMEM, block_shape=(1, 1),
                         index_map=lambda i: (0, 0)),
        ],
        out_shape=[
            jax.ShapeDtypeStruct((B, H1), jnp.float32),
            jax.ShapeDtypeStruct((B, H2), jnp.float32),
            jax.ShapeDtypeStruct((B, H3), jnp.float32),
            jax.ShapeDtypeStruct((B, 1), jnp.float32),
            jax.ShapeDtypeStruct((1, 1), jnp.float32),
        ],
    )(gmf, emu, emi, lab2, W1a, W1b, b1, W2, b2, W3, b3, Wpa, Wpb, bp)


def kernel(user, item, label, eu_gmf, ei_gmf, eu_mlp, ei_mlp,
           W1, b1, W2, b2, W3, b3, Wp, bp):
    uhi = lax.shift_right_logical(user, 7)
    ulo = jnp.bitwise_and(user, 127)
    ihi = lax.shift_right_logical(item, 7)
    ilo = jnp.bitwise_and(item, 127)
    gmf, emu, emi = _sc_gather(uhi, ulo, ihi, ilo,
                               eu_gmf.T, ei_gmf.T, eu_mlp.T, ei_mlp.T)
    W1a, W1b = W1[:, :DM], W1[:, DM:]
    Wpa, Wpb = Wp[:, :DG], Wp[:, DG:]
    m1, m2, m3, pred2, loss = _tc_mlp(
        gmf, emu, emi, label.reshape(B, 1),
        W1a, W1b, b1.reshape(1, H1), W2, b2.reshape(1, H2),
        W3, b3.reshape(1, H3), Wpa, Wpb, bp.reshape(1, 1),
    )
    return (loss[0, 0], m1, m2, m3, pred2.reshape(-1))
